# baseline (device time: 21723 ns/iter reference)
import jax
import jax.numpy as jnp
from jax import lax
from jax.experimental import pallas as pl
from jax.experimental.pallas import tpu as pltpu

N_DEV = 16
P = N_DEV - 1


def kernel(x):
    _, m, n = x.shape
    c_rows = m // N_DEV
    h_rows = c_rows // 2

    def body(x_ref, out_ref, acc_ref, rbuf, send_sems, recv_sems):
        i = lax.axis_index("i")

        acc_ref[...] = x_ref[0].astype(jnp.bfloat16)

        barrier_sem = pltpu.get_barrier_semaphore()
        for d in range(1, N_DEV):
            pl.semaphore_signal(
                barrier_sem, inc=1,
                device_id=((i + d) % N_DEV,),
                device_id_type=pl.DeviceIdType.MESH,
            )
        pl.semaphore_wait(barrier_sem, P)

        all_sends = []

        def dummy_recv(buf, slot):
            return pltpu.make_async_remote_copy(
                src_ref=buf, dst_ref=buf,
                send_sem=send_sems.at[slot], recv_sem=recv_sems.at[slot],
                device_id=(i,), device_id_type=pl.DeviceIdType.MESH,
            )

        my_off = i * c_rows

        for h in (0, 1):
            for d in range(1, N_DEV):
                pt = (i + d) % N_DEV
                r = pltpu.make_async_remote_copy(
                    src_ref=acc_ref.at[pl.ds(pt * c_rows + h * h_rows, h_rows)],
                    dst_ref=rbuf.at[h, P - d],
                    send_sem=send_sems.at[15 * h + d - 1],
                    recv_sem=recv_sems.at[15 * h + P - d],
                    device_id=(pt,),
                    device_id_type=pl.DeviceIdType.MESH,
                )
                r.start()
                all_sends.append(r)

        def reduce_half(h):
            off = my_off + h * h_rows
            for group in ((14, 13, 12, 11, 10), (9, 8, 7, 6, 5),
                          (4, 3, 2, 1, 0)):
                for slot in group:
                    dummy_recv(rbuf.at[h, slot], 15 * h + slot).wait_recv()
                acc_ref[pl.ds(off, h_rows)] = (
                    acc_ref[pl.ds(off, h_rows)]
                    + rbuf[h, group[0]] + rbuf[h, group[1]]
                    + rbuf[h, group[2]] + rbuf[h, group[3]]
                    + rbuf[h, group[4]]
                )

        def broadcast_half(h):
            off = my_off + h * h_rows
            for d in range(1, N_DEV):
                pt = (i + d) % N_DEV
                r = pltpu.make_async_remote_copy(
                    src_ref=acc_ref.at[pl.ds(off, h_rows)],
                    dst_ref=acc_ref.at[pl.ds(off, h_rows)],
                    send_sem=send_sems.at[30 + 15 * h + d - 1],
                    recv_sem=recv_sems.at[30 + 15 * h + P - d],
                    device_id=(pt,),
                    device_id_type=pl.DeviceIdType.MESH,
                )
                r.start()
                all_sends.append(r)

        reduce_half(0)
        broadcast_half(0)
        reduce_half(1)
        broadcast_half(1)

        out_ref[pl.ds(my_off, c_rows)] = acc_ref[pl.ds(my_off, c_rows)]
        for s in range(P - 1, -1, -1):
            dummy_recv(acc_ref.at[pl.ds(my_off, h_rows)], 30 + s).wait_recv()
            dummy_recv(acc_ref.at[pl.ds(my_off, h_rows)], 45 + s).wait_recv()
            off = ((i - (P - s)) % N_DEV) * c_rows
            out_ref[pl.ds(off, c_rows)] = acc_ref[pl.ds(off, c_rows)]

        for r in all_sends:
            r.wait_send()

    return pl.pallas_call(
        body,
        out_shape=jax.ShapeDtypeStruct((m, n), jnp.bfloat16),
        in_specs=[pl.BlockSpec(memory_space=pltpu.VMEM)],
        out_specs=pl.BlockSpec(memory_space=pltpu.VMEM),
        scratch_shapes=[
            pltpu.VMEM((m, n), jnp.bfloat16),
            pltpu.VMEM((2, P, h_rows, n), jnp.bfloat16),
            pltpu.SemaphoreType.DMA((60,)),
            pltpu.SemaphoreType.DMA((60,)),
        ],
        compiler_params=pltpu.CompilerParams(collective_id=0),
    )(x)
